# baseline (speedup 1.0000x reference)
import functools

import jax
import jax.numpy as jnp
from jax import lax
from jax.experimental import pallas as pl
from jax.experimental.pallas import tpu as pltpu
from jax.experimental.pallas import tpu_sc as plsc

_B, _P, _K = 32, 1024, 16
_N = _B * _P
_E = _N * _K
_NUM_CLASSES = 40
_RB = 512


def _knn_body(posr_ref, posc_ref, idx_ref, d2_ref):
    b = pl.program_id(0)
    pr = posr_ref[...]
    pc = posc_ref[...]
    sqr = jnp.sum(pr * pr, axis=1, keepdims=True)
    sqc = jnp.sum(pc * pc, axis=1)
    dots = lax.dot_general(pr, pc, (((1,), (1,)), ((), ())),
                           preferred_element_type=jnp.float32)
    d2_ref[...] = sqr + sqc[None, :] - 2.0 * dots
    colids = lax.broadcasted_iota(jnp.int32, (_RB, _P), 1)
    lanek = lax.broadcasted_iota(jnp.int32, (_RB, _K), 1)
    inf = jnp.float32(jnp.inf)

    def body(k, sels):
        d2 = d2_ref[...]
        m = jnp.min(d2, axis=1, keepdims=True)
        cand = jnp.where(d2 == m, colids, _P)
        sel = jnp.min(cand, axis=1, keepdims=True)
        d2_ref[...] = jnp.where(colids == sel, inf, d2)
        return jnp.where(lanek == k, sel, sels)

    sels = lax.fori_loop(0, _K, body, jnp.zeros((_RB, _K), jnp.int32))
    idx_ref[...] = sels + b * _P


def _knn_call(pos, interpret=False):
    nr = _P // _RB
    return pl.pallas_call(
        _knn_body,
        grid=(_B, nr),
        in_specs=[
            pl.BlockSpec((_RB, 3), lambda b, r: (b * nr + r, 0)),
            pl.BlockSpec((_P, 3), lambda b, r: (b, 0)),
        ],
        out_specs=pl.BlockSpec((_RB, _K), lambda b, r: (b * nr + r, 0)),
        out_shape=jax.ShapeDtypeStruct((_N, _K), jnp.int32),
        scratch_shapes=[pltpu.VMEM((_RB, _P), jnp.float32)],
        interpret=interpret,
    )(pos, pos)


def _sc_gather(table, idx, chunk=2048):
    _, d = table.shape
    m = idx.shape[0]
    nw = 32
    per_w = m // nw
    nch = per_w // chunk
    assert per_w % chunk == 0 and d % 16 == 0
    mesh = plsc.VectorSubcoreMesh(core_axis_name="c", subcore_axis_name="s")

    @functools.partial(
        pl.kernel, mesh=mesh,
        out_type=jax.ShapeDtypeStruct((m, d), jnp.float32),
        scratch_types=[
            pltpu.VMEM((chunk,), jnp.int32),
            pltpu.VMEM((chunk, d), jnp.float32),
            pltpu.SemaphoreType.DMA,
        ],
        compiler_params=pltpu.CompilerParams(use_tc_tiling_on_sc=False),
    )
    def k(table_hbm, idx_hbm, out_hbm, idx_v, rows_v, sem):
        wid = lax.axis_index("s") * 2 + lax.axis_index("c")

        def body(c, carry):
            base = wid * per_w + c * chunk
            pltpu.sync_copy(idx_hbm.at[pl.ds(base, chunk)], idx_v)
            pltpu.async_copy(table_hbm.at[idx_v], rows_v, sem).wait()
            pltpu.sync_copy(rows_v, out_hbm.at[pl.ds(base, chunk)])
            return carry

        lax.fori_loop(0, nch, body, 0)

    return k(table, idx)


def _psqrt(x):
    r = lax.rsqrt(x)
    r = r * (1.5 - 0.5 * x * r * r)
    r = r * (1.5 - 0.5 * x * r * r)
    return x * r

_ATAN_C = (0.9999999880821423, -0.333331207776302, 0.1999371609301849,
           -0.14213195870325876, 0.10681419898649704, -0.07596807640005956,
           0.04385557421611047, -0.016827433863723106, 0.003049964511658644)


def _atan2_nn(y, x):
    ax = jnp.abs(x)
    num = jnp.minimum(y, ax)
    den = jnp.maximum(y, ax)
    r = lax.reciprocal(den)
    r = r * (2.0 - den * r)
    r = r * (2.0 - den * r)
    q = num * r
    s = q * q
    acc = jnp.float32(_ATAN_C[-1])
    for c in _ATAN_C[-2::-1]:
        acc = acc * s + jnp.float32(c)
    base = acc * q
    res = jnp.where(y > ax, jnp.float32(jnp.pi / 2) - base, base)
    return jnp.where(x < 0, jnp.float32(jnp.pi) - res, res)


def _angle(v1x, v1y, v1z, v2x, v2y, v2z):
    cx = v1y * v2z - v1z * v2y
    cy = v1z * v2x - v1x * v2z
    cz = v1x * v2y - v1y * v2x
    cn2 = cx * cx + cy * cy + cz * cz
    cn = jnp.where(cn2 == 0, 0.0, _psqrt(jnp.where(cn2 == 0, 1.0, cn2)))
    dot = v1x * v2x + v1y * v2y + v1z * v2z
    both = (cn == 0) & (dot == 0)
    return _atan2_nn(jnp.where(both, 0.0, cn), jnp.where(both, 1.0, dot))


def _ppf_rows(git, gjt):
    dx = gjt[0:1, :] - git[0:1, :]
    dy = gjt[1:2, :] - git[1:2, :]
    dz = gjt[2:3, :] - git[2:3, :]
    nix, niy, niz = git[3:4, :], git[4:5, :], git[5:6, :]
    njx, njy, njz = gjt[3:4, :], gjt[4:5, :], gjt[5:6, :]
    d2s = dx * dx + dy * dy + dz * dz
    dn = jnp.where(d2s == 0, 0.0, _psqrt(jnp.where(d2s == 0, 1.0, d2s)))
    a1 = _angle(nix, niy, niz, dx, dy, dz)
    a2 = _angle(njx, njy, njz, dx, dy, dz)
    a3 = _angle(nix, niy, niz, njx, njy, njz)
    return dn, a1, a2, a3


def _conv1_body(gj_ref, git_ref, w0_ref, b0_ref, w1_ref, b1_ref,
                x_ref, ft_ref, acc_ref):
    k = pl.program_id(1)
    git = git_ref[...]
    gjt = jnp.transpose(gj_ref[...])
    dn, a1, a2, a3 = _ppf_rows(git, gjt)
    ft = jnp.concatenate([dn, a1, a2, a3], axis=0)
    ft_ref[0] = ft
    h = jax.nn.relu(
        lax.dot_general(ft, w0_ref[...], (((0,), (0,)), ((), ())),
                        preferred_element_type=jnp.float32) + b0_ref[...])
    h = jax.nn.relu(
        lax.dot_general(h, w1_ref[...], (((1,), (0,)), ((), ())),
                        preferred_element_type=jnp.float32) + b1_ref[...])
    prev = acc_ref[...]
    new = jnp.where(k == 0, h, jnp.maximum(prev, h))
    acc_ref[...] = new

    @pl.when(k == _K - 1)
    def _():
        x_ref[...] = jax.nn.relu(new)


def _conv1_call(g, table_t, m1w0, m1b0, m1w1, m1b1, interpret=False):
    return pl.pallas_call(
        _conv1_body,
        grid=(_B, _K),
        in_specs=[
            pl.BlockSpec((_P, 16), lambda b, k: (k * _B + b, 0)),
            pl.BlockSpec((16, _P), lambda b, k: (0, b)),
            pl.BlockSpec((4, 16), lambda b, k: (0, 0)),
            pl.BlockSpec((1, 16), lambda b, k: (0, 0)),
            pl.BlockSpec((16, 32), lambda b, k: (0, 0)),
            pl.BlockSpec((1, 32), lambda b, k: (0, 0)),
        ],
        out_specs=[
            pl.BlockSpec((_P, 32), lambda b, k: (b, 0)),
            pl.BlockSpec((1, 4, _P), lambda b, k: (k, 0, b)),
        ],
        out_shape=[
            jax.ShapeDtypeStruct((_N, 32), jnp.float32),
            jax.ShapeDtypeStruct((_K, 4, _N), jnp.float32),
        ],
        scratch_shapes=[pltpu.VMEM((_P, 32), jnp.float32)],
        interpret=interpret,
    )(g, table_t, m1w0, m1b0.reshape(1, 16), m1w1, m1b1.reshape(1, 32))


def _conv2_body(xg_ref, ft_ref, w0_ref, b0_ref, w1_ref, b1_ref,
                cw_ref, cb_ref, out_ref, acc_ref):
    k = pl.program_id(1)
    w0 = w0_ref[...]
    h = (lax.dot_general(xg_ref[...], w0[0:32, :], (((1,), (0,)), ((), ())),
                         preferred_element_type=jnp.float32)
         + lax.dot_general(ft_ref[0], w0[32:36, :], (((0,), (0,)), ((), ())),
                           preferred_element_type=jnp.float32)
         + b0_ref[...])
    h = jax.nn.relu(h)
    h = jax.nn.relu(
        lax.dot_general(h, w1_ref[...], (((1,), (0,)), ((), ())),
                        preferred_element_type=jnp.float32) + b1_ref[...])
    prev = acc_ref[...]
    new = jnp.where(k == 0, h, jnp.maximum(prev, h))
    acc_ref[...] = new

    @pl.when(k == _K - 1)
    def _():
        x2 = jax.nn.relu(new)
        pooled = jnp.max(x2, axis=0, keepdims=True)
        out_ref[0] = (lax.dot_general(pooled, cw_ref[...],
                                      (((1,), (0,)), ((), ())),
                                      preferred_element_type=jnp.float32)
                      + cb_ref[...])


def _conv2_call(xg, ft, m2w0, m2b0, m2w1, m2b1, cw, cb, interpret=False):
    return pl.pallas_call(
        _conv2_body,
        grid=(_B, _K),
        in_specs=[
            pl.BlockSpec((_P, 32), lambda b, k: (k * _B + b, 0)),
            pl.BlockSpec((1, 4, _P), lambda b, k: (k, 0, b)),
            pl.BlockSpec((36, 64), lambda b, k: (0, 0)),
            pl.BlockSpec((1, 64), lambda b, k: (0, 0)),
            pl.BlockSpec((64, 128), lambda b, k: (0, 0)),
            pl.BlockSpec((1, 128), lambda b, k: (0, 0)),
            pl.BlockSpec((128, _NUM_CLASSES), lambda b, k: (0, 0)),
            pl.BlockSpec((1, _NUM_CLASSES), lambda b, k: (0, 0)),
        ],
        out_specs=pl.BlockSpec((1, 1, _NUM_CLASSES), lambda b, k: (b, 0, 0)),
        out_shape=jax.ShapeDtypeStruct((_B, 1, _NUM_CLASSES), jnp.float32),
        scratch_shapes=[pltpu.VMEM((_P, 128), jnp.float32)],
        interpret=interpret,
    )(xg, ft, m2w0, m2b0.reshape(1, 64), m2w1, m2b1.reshape(1, 128),
      cw, cb.reshape(1, _NUM_CLASSES)).reshape(_B, _NUM_CLASSES)


def kernel(pos, normal, batch, m1w0, m1b0, m1w1, m1b1, m2w0, m2b0, m2w1, m2b1,
           cw, cb):
    del batch
    idx = _knn_call(pos)
    src_km = idx.T.reshape(_E)
    table = jnp.concatenate(
        [pos, normal, jnp.zeros((_N, 10), jnp.float32)], axis=1)
    table_t = jnp.concatenate([pos.T, normal.T], axis=0)
    table_t = jnp.concatenate(
        [table_t, jnp.zeros((10, _N), jnp.float32)], axis=0)
    g = _sc_gather(table, src_km)
    x, ft = _conv1_call(g, table_t, m1w0, m1b0, m1w1, m1b1)
    xg = _sc_gather(x, src_km)
    return _conv2_call(xg, ft, m2w0, m2b0, m2w1, m2b1, cw, cb)


# speedup vs baseline: 1.0219x; 1.0219x over previous
import functools

import jax
import jax.numpy as jnp
from jax import lax
from jax.experimental import pallas as pl
from jax.experimental.pallas import tpu as pltpu
from jax.experimental.pallas import tpu_sc as plsc

_B, _P, _K = 32, 1024, 16
_N = _B * _P
_E = _N * _K
_NUM_CLASSES = 40
_RB = 512


def _knn_body(posr_ref, posc_ref, idx_ref, d2_ref):
    b = pl.program_id(0)
    pr = posr_ref[...]
    pc = posc_ref[...]
    sqr = jnp.sum(pr * pr, axis=1, keepdims=True)
    sqc = jnp.sum(pc * pc, axis=1)
    dots = lax.dot_general(pr, pc, (((1,), (1,)), ((), ())),
                           preferred_element_type=jnp.float32)
    d2_ref[...] = sqr + sqc[None, :] - 2.0 * dots
    colids = lax.broadcasted_iota(jnp.int32, (_RB, _P), 1)
    lanek = lax.broadcasted_iota(jnp.int32, (_RB, _K), 1)
    inf = jnp.float32(jnp.inf)

    def body(k, sels):
        d2 = d2_ref[...]
        m = jnp.min(d2, axis=1, keepdims=True)
        cand = jnp.where(d2 == m, colids, _P)
        sel = jnp.min(cand, axis=1, keepdims=True)
        d2_ref[...] = jnp.where(colids == sel, inf, d2)
        return jnp.where(lanek == k, sel, sels)

    sels = lax.fori_loop(0, _K, body, jnp.zeros((_RB, _K), jnp.int32))
    idx_ref[...] = sels + b * _P


def _knn_call(pos, interpret=False):
    nr = _P // _RB
    return pl.pallas_call(
        _knn_body,
        grid=(_B, nr),
        in_specs=[
            pl.BlockSpec((_RB, 3), lambda b, r: (b * nr + r, 0)),
            pl.BlockSpec((_P, 3), lambda b, r: (b, 0)),
        ],
        out_specs=pl.BlockSpec((_RB, _K), lambda b, r: (b * nr + r, 0)),
        out_shape=jax.ShapeDtypeStruct((_N, _K), jnp.int32),
        scratch_shapes=[pltpu.VMEM((_RB, _P), jnp.float32)],
        interpret=interpret,
    )(pos, pos)


def _sc_gather(table, idx, chunk=2048):
    _, d = table.shape
    m = idx.shape[0]
    nw = 32
    per_w = m // nw
    nch = per_w // chunk
    assert per_w % chunk == 0 and d % 16 == 0
    mesh = plsc.VectorSubcoreMesh(core_axis_name="c", subcore_axis_name="s")

    @functools.partial(
        pl.kernel, mesh=mesh,
        out_type=jax.ShapeDtypeStruct((m, d), jnp.float32),
        scratch_types=[
            pltpu.VMEM((chunk,), jnp.int32),
            pltpu.VMEM((chunk, d), jnp.float32),
            pltpu.SemaphoreType.DMA,
        ],
        compiler_params=pltpu.CompilerParams(use_tc_tiling_on_sc=False),
    )
    def k(table_hbm, idx_hbm, out_hbm, idx_v, rows_v, sem):
        wid = lax.axis_index("s") * 2 + lax.axis_index("c")

        def body(c, carry):
            base = wid * per_w + c * chunk
            pltpu.sync_copy(idx_hbm.at[pl.ds(base, chunk)], idx_v)
            pltpu.async_copy(table_hbm.at[idx_v], rows_v, sem).wait()
            pltpu.sync_copy(rows_v, out_hbm.at[pl.ds(base, chunk)])
            return carry

        lax.fori_loop(0, nch, body, 0)

    return k(table, idx)


def _psqrt(x):
    r = lax.rsqrt(x)
    r = r * (1.5 - 0.5 * x * r * r)
    r = r * (1.5 - 0.5 * x * r * r)
    return x * r

_ATAN_C = (0.9999999880821423, -0.333331207776302, 0.1999371609301849,
           -0.14213195870325876, 0.10681419898649704, -0.07596807640005956,
           0.04385557421611047, -0.016827433863723106, 0.003049964511658644)


def _atan2_nn(y, x):
    ax = jnp.abs(x)
    num = jnp.minimum(y, ax)
    den = jnp.maximum(y, ax)
    r = lax.reciprocal(den)
    r = r * (2.0 - den * r)
    r = r * (2.0 - den * r)
    q = num * r
    s = q * q
    acc = jnp.float32(_ATAN_C[-1])
    for c in _ATAN_C[-2::-1]:
        acc = acc * s + jnp.float32(c)
    base = acc * q
    res = jnp.where(y > ax, jnp.float32(jnp.pi / 2) - base, base)
    return jnp.where(x < 0, jnp.float32(jnp.pi) - res, res)


def _angle(v1x, v1y, v1z, v2x, v2y, v2z):
    cx = v1y * v2z - v1z * v2y
    cy = v1z * v2x - v1x * v2z
    cz = v1x * v2y - v1y * v2x
    cn2 = cx * cx + cy * cy + cz * cz
    cn = jnp.where(cn2 == 0, 0.0, _psqrt(jnp.where(cn2 == 0, 1.0, cn2)))
    dot = v1x * v2x + v1y * v2y + v1z * v2z
    both = (cn == 0) & (dot == 0)
    return _atan2_nn(jnp.where(both, 0.0, cn), jnp.where(both, 1.0, dot))


def _ppf_rows(git, gjt):
    dx = gjt[0:1, :] - git[0:1, :]
    dy = gjt[1:2, :] - git[1:2, :]
    dz = gjt[2:3, :] - git[2:3, :]
    nix, niy, niz = git[3:4, :], git[4:5, :], git[5:6, :]
    njx, njy, njz = gjt[3:4, :], gjt[4:5, :], gjt[5:6, :]
    d2s = dx * dx + dy * dy + dz * dz
    dn = jnp.where(d2s == 0, 0.0, _psqrt(jnp.where(d2s == 0, 1.0, d2s)))
    a1 = _angle(nix, niy, niz, dx, dy, dz)
    a2 = _angle(njx, njy, njz, dx, dy, dz)
    a3 = _angle(nix, niy, niz, njx, njy, njz)
    return dn, a1, a2, a3


def _conv1_body(gj_ref, git_ref, w0_ref, b0_ref, w1_ref, b1_ref,
                x_ref, ft_ref, acc_ref):
    k = pl.program_id(1)
    git = git_ref[...]
    gjt = gj_ref[0]
    dn, a1, a2, a3 = _ppf_rows(git, gjt)
    ft = jnp.concatenate([dn, a1, a2, a3], axis=0)
    ft_ref[0] = ft
    h = jax.nn.relu(
        lax.dot_general(ft, w0_ref[...], (((0,), (0,)), ((), ())),
                        preferred_element_type=jnp.float32) + b0_ref[...])
    h = jax.nn.relu(
        lax.dot_general(h, w1_ref[...], (((1,), (0,)), ((), ())),
                        preferred_element_type=jnp.float32) + b1_ref[...])
    prev = acc_ref[...]
    new = jnp.where(k == 0, h, jnp.maximum(prev, h))
    acc_ref[...] = new

    @pl.when(k == _K - 1)
    def _():
        x_ref[...] = jax.nn.relu(new)


def _conv1_call(g, table_t, m1w0, m1b0, m1w1, m1b1, interpret=False):
    return pl.pallas_call(
        _conv1_body,
        grid=(_B, _K),
        in_specs=[
            pl.BlockSpec((1, 16, _P), lambda b, k: (k, 0, b)),
            pl.BlockSpec((16, _P), lambda b, k: (0, b)),
            pl.BlockSpec((4, 16), lambda b, k: (0, 0)),
            pl.BlockSpec((1, 16), lambda b, k: (0, 0)),
            pl.BlockSpec((16, 32), lambda b, k: (0, 0)),
            pl.BlockSpec((1, 32), lambda b, k: (0, 0)),
        ],
        out_specs=[
            pl.BlockSpec((_P, 32), lambda b, k: (b, 0)),
            pl.BlockSpec((1, 4, _P), lambda b, k: (k, 0, b)),
        ],
        out_shape=[
            jax.ShapeDtypeStruct((_N, 32), jnp.float32),
            jax.ShapeDtypeStruct((_K, 4, _N), jnp.float32),
        ],
        scratch_shapes=[pltpu.VMEM((_P, 32), jnp.float32)],
        interpret=interpret,
    )(g, table_t, m1w0, m1b0.reshape(1, 16), m1w1, m1b1.reshape(1, 32))


def _conv2_body(xg_ref, ft_ref, w0_ref, b0_ref, w1_ref, b1_ref,
                cw_ref, cb_ref, out_ref, acc_ref):
    k = pl.program_id(1)
    w0 = w0_ref[...]
    h = (lax.dot_general(xg_ref[...], w0[0:32, :], (((1,), (0,)), ((), ())),
                         preferred_element_type=jnp.float32)
         + lax.dot_general(ft_ref[0], w0[32:36, :], (((0,), (0,)), ((), ())),
                           preferred_element_type=jnp.float32)
         + b0_ref[...])
    h = jax.nn.relu(h)
    h = jax.nn.relu(
        lax.dot_general(h, w1_ref[...], (((1,), (0,)), ((), ())),
                        preferred_element_type=jnp.float32) + b1_ref[...])
    prev = acc_ref[...]
    new = jnp.where(k == 0, h, jnp.maximum(prev, h))
    acc_ref[...] = new

    @pl.when(k == _K - 1)
    def _():
        x2 = jax.nn.relu(new)
        pooled = jnp.max(x2, axis=0, keepdims=True)
        out_ref[0] = (lax.dot_general(pooled, cw_ref[...],
                                      (((1,), (0,)), ((), ())),
                                      preferred_element_type=jnp.float32)
                      + cb_ref[...])


def _conv2_call(xg, ft, m2w0, m2b0, m2w1, m2b1, cw, cb, interpret=False):
    return pl.pallas_call(
        _conv2_body,
        grid=(_B, _K),
        in_specs=[
            pl.BlockSpec((_P, 32), lambda b, k: (k * _B + b, 0)),
            pl.BlockSpec((1, 4, _P), lambda b, k: (k, 0, b)),
            pl.BlockSpec((36, 64), lambda b, k: (0, 0)),
            pl.BlockSpec((1, 64), lambda b, k: (0, 0)),
            pl.BlockSpec((64, 128), lambda b, k: (0, 0)),
            pl.BlockSpec((1, 128), lambda b, k: (0, 0)),
            pl.BlockSpec((128, _NUM_CLASSES), lambda b, k: (0, 0)),
            pl.BlockSpec((1, _NUM_CLASSES), lambda b, k: (0, 0)),
        ],
        out_specs=pl.BlockSpec((1, 1, _NUM_CLASSES), lambda b, k: (b, 0, 0)),
        out_shape=jax.ShapeDtypeStruct((_B, 1, _NUM_CLASSES), jnp.float32),
        scratch_shapes=[pltpu.VMEM((_P, 128), jnp.float32)],
        interpret=interpret,
    )(xg, ft, m2w0, m2b0.reshape(1, 64), m2w1, m2b1.reshape(1, 128),
      cw, cb.reshape(1, _NUM_CLASSES)).reshape(_B, _NUM_CLASSES)


def kernel(pos, normal, batch, m1w0, m1b0, m1w1, m1b1, m2w0, m2b0, m2w1, m2b1,
           cw, cb):
    del batch
    idx = _knn_call(pos)
    src_km = idx.T.reshape(_E)
    table = jnp.concatenate(
        [pos, normal, jnp.zeros((_N, 10), jnp.float32)], axis=1)
    table_t = jnp.concatenate([pos.T, normal.T], axis=0)
    table_t = jnp.concatenate(
        [table_t, jnp.zeros((10, _N), jnp.float32)], axis=0)
    g = _sc_gather(table, src_km, chunk=4096)
    g_t = g.reshape(_K, _N, 16).transpose(0, 2, 1)
    x, ft = _conv1_call(g_t, table_t, m1w0, m1b0, m1w1, m1b1)
    xg = _sc_gather(x, src_km)
    return _conv2_call(xg, ft, m2w0, m2b0, m2w1, m2b1, cw, cb)


# speedup vs baseline: 1.1890x; 1.1636x over previous
import functools

import jax
import jax.numpy as jnp
from jax import lax
from jax.experimental import pallas as pl
from jax.experimental.pallas import tpu as pltpu
from jax.experimental.pallas import tpu_sc as plsc

_B, _P, _K = 32, 1024, 16
_N = _B * _P
_E = _N * _K
_NUM_CLASSES = 40
_RB = 512


def _knn_body(posr_ref, posc_ref, idx_ref, d2_ref):
    b = pl.program_id(0)
    pr = posr_ref[...]
    pc = posc_ref[...]
    sqr = jnp.sum(pr * pr, axis=1, keepdims=True)
    sqc = jnp.sum(pc * pc, axis=1)
    dots = lax.dot_general(pr, pc, (((1,), (1,)), ((), ())),
                           preferred_element_type=jnp.float32)
    d2_ref[...] = sqr + sqc[None, :] - 2.0 * dots
    colids = lax.broadcasted_iota(jnp.int32, (_RB, _P), 1)
    lanek = lax.broadcasted_iota(jnp.int32, (_RB, _K), 1)
    inf = jnp.float32(jnp.inf)

    def body(k, sels):
        d2 = d2_ref[...]
        m = jnp.min(d2, axis=1, keepdims=True)
        cand = jnp.where(d2 == m, colids, _P)
        sel = jnp.min(cand, axis=1, keepdims=True)
        d2_ref[...] = jnp.where(colids == sel, inf, d2)
        return jnp.where(lanek == k, sel, sels)

    sels = lax.fori_loop(0, _K, body, jnp.zeros((_RB, _K), jnp.int32))
    idx_ref[...] = sels + b * _P


def _knn_call(pos, interpret=False):
    nr = _P // _RB
    return pl.pallas_call(
        _knn_body,
        grid=(_B, nr),
        in_specs=[
            pl.BlockSpec((_RB, 3), lambda b, r: (b * nr + r, 0)),
            pl.BlockSpec((_P, 3), lambda b, r: (b, 0)),
        ],
        out_specs=pl.BlockSpec((_RB, _K), lambda b, r: (b * nr + r, 0)),
        out_shape=jax.ShapeDtypeStruct((_N, _K), jnp.int32),
        scratch_shapes=[pltpu.VMEM((_RB, _P), jnp.float32)],
        interpret=interpret,
    )(pos, pos)


def _sc_gather(table, idx, chunk=2048):
    _, d = table.shape
    m = idx.shape[0]
    nw = 32
    per_w = m // nw
    nch = per_w // chunk
    assert per_w % chunk == 0 and d % 16 == 0
    mesh = plsc.VectorSubcoreMesh(core_axis_name="c", subcore_axis_name="s")

    @functools.partial(
        pl.kernel, mesh=mesh,
        out_type=jax.ShapeDtypeStruct((m, d), jnp.float32),
        scratch_types=[
            pltpu.VMEM((chunk,), jnp.int32),
            pltpu.VMEM((chunk, d), jnp.float32),
            pltpu.SemaphoreType.DMA,
        ],
        compiler_params=pltpu.CompilerParams(use_tc_tiling_on_sc=False),
    )
    def k(table_hbm, idx_hbm, out_hbm, idx_v, rows_v, sem):
        wid = lax.axis_index("s") * 2 + lax.axis_index("c")

        def body(c, carry):
            base = wid * per_w + c * chunk
            pltpu.sync_copy(idx_hbm.at[pl.ds(base, chunk)], idx_v)
            pltpu.async_copy(table_hbm.at[idx_v], rows_v, sem).wait()
            pltpu.sync_copy(rows_v, out_hbm.at[pl.ds(base, chunk)])
            return carry

        lax.fori_loop(0, nch, body, 0)

    return k(table, idx)


def _psqrt(x):
    r = lax.rsqrt(x)
    r = r * (1.5 - 0.5 * x * r * r)
    r = r * (1.5 - 0.5 * x * r * r)
    return x * r

_ATAN_C = (0.9999999880821423, -0.333331207776302, 0.1999371609301849,
           -0.14213195870325876, 0.10681419898649704, -0.07596807640005956,
           0.04385557421611047, -0.016827433863723106, 0.003049964511658644)


def _atan2_nn(y, x):
    ax = jnp.abs(x)
    num = jnp.minimum(y, ax)
    den = jnp.maximum(y, ax)
    r = lax.reciprocal(den)
    r = r * (2.0 - den * r)
    r = r * (2.0 - den * r)
    q = num * r
    s = q * q
    acc = jnp.float32(_ATAN_C[-1])
    for c in _ATAN_C[-2::-1]:
        acc = acc * s + jnp.float32(c)
    base = acc * q
    res = jnp.where(y > ax, jnp.float32(jnp.pi / 2) - base, base)
    return jnp.where(x < 0, jnp.float32(jnp.pi) - res, res)


def _angle(v1x, v1y, v1z, v2x, v2y, v2z):
    cx = v1y * v2z - v1z * v2y
    cy = v1z * v2x - v1x * v2z
    cz = v1x * v2y - v1y * v2x
    cn2 = cx * cx + cy * cy + cz * cz
    cn = jnp.where(cn2 == 0, 0.0, _psqrt(jnp.where(cn2 == 0, 1.0, cn2)))
    dot = v1x * v2x + v1y * v2y + v1z * v2z
    both = (cn == 0) & (dot == 0)
    return _atan2_nn(jnp.where(both, 0.0, cn), jnp.where(both, 1.0, dot))


def _ppf_rows(git, gjt):
    dx = gjt[0:1, :] - git[0:1, :]
    dy = gjt[1:2, :] - git[1:2, :]
    dz = gjt[2:3, :] - git[2:3, :]
    nix, niy, niz = git[3:4, :], git[4:5, :], git[5:6, :]
    njx, njy, njz = gjt[3:4, :], gjt[4:5, :], gjt[5:6, :]
    d2s = dx * dx + dy * dy + dz * dz
    dn = jnp.where(d2s == 0, 0.0, _psqrt(jnp.where(d2s == 0, 1.0, d2s)))
    a1 = _angle(nix, niy, niz, dx, dy, dz)
    a2 = _angle(njx, njy, njz, dx, dy, dz)
    a3 = _angle(nix, niy, niz, njx, njy, njz)
    return dn, a1, a2, a3


def _conv1_body(gj0_ref, gj1_ref, git_ref, w0_ref, b0_ref, w1_ref, b1_ref,
                x_ref, ft_ref, acc_ref):
    j = pl.program_id(1)
    git = git_ref[...]
    fts = []
    for gj_ref in (gj0_ref, gj1_ref):
        dn, a1, a2, a3 = _ppf_rows(git, gj_ref[0])
        fts.append(jnp.concatenate([dn, a1, a2, a3], axis=0))
    ft_ref[0, 0] = fts[0]
    ft_ref[0, 1] = fts[1]
    ftcat = jnp.concatenate(fts, axis=1)
    h = jax.nn.relu(
        lax.dot_general(ftcat, w0_ref[...], (((0,), (0,)), ((), ())),
                        preferred_element_type=jnp.float32) + b0_ref[...])
    h = jax.nn.relu(
        lax.dot_general(h, w1_ref[...], (((1,), (0,)), ((), ())),
                        preferred_element_type=jnp.float32) + b1_ref[...])
    hm = jnp.maximum(h[0:_P, :], h[_P:2 * _P, :])
    prev = acc_ref[...]
    new = jnp.where(j == 0, hm, jnp.maximum(prev, hm))
    acc_ref[...] = new

    @pl.when(j == _K // 2 - 1)
    def _():
        x_ref[...] = jax.nn.relu(new)


def _conv1_call(g, table_t, m1w0, m1b0, m1w1, m1b1, interpret=False):
    return pl.pallas_call(
        _conv1_body,
        grid=(_B, _K // 2),
        in_specs=[
            pl.BlockSpec((1, 16, _P), lambda b, j: (2 * j, 0, b)),
            pl.BlockSpec((1, 16, _P), lambda b, j: (2 * j + 1, 0, b)),
            pl.BlockSpec((16, _P), lambda b, j: (0, b)),
            pl.BlockSpec((4, 16), lambda b, j: (0, 0)),
            pl.BlockSpec((1, 16), lambda b, j: (0, 0)),
            pl.BlockSpec((16, 32), lambda b, j: (0, 0)),
            pl.BlockSpec((1, 32), lambda b, j: (0, 0)),
        ],
        out_specs=[
            pl.BlockSpec((_P, 32), lambda b, j: (b, 0)),
            pl.BlockSpec((1, 2, 4, _P), lambda b, j: (j, 0, 0, b)),
        ],
        out_shape=[
            jax.ShapeDtypeStruct((_N, 32), jnp.float32),
            jax.ShapeDtypeStruct((_K // 2, 2, 4, _N), jnp.float32),
        ],
        scratch_shapes=[pltpu.VMEM((_P, 32), jnp.float32)],
        interpret=interpret,
    )(g, g, table_t, m1w0, m1b0.reshape(1, 16), m1w1, m1b1.reshape(1, 32))


def _conv2_body(xg0_ref, xg1_ref, ft_ref, w0_ref, b0_ref, w1_ref, b1_ref,
                cw_ref, cb_ref, out_ref, acc_ref):
    j = pl.program_id(1)
    w0 = w0_ref[...]
    xgcat = jnp.concatenate([xg0_ref[...], xg1_ref[...]], axis=0)
    ftcat = jnp.concatenate([ft_ref[0, 0], ft_ref[0, 1]], axis=1)
    h = (lax.dot_general(xgcat, w0[0:32, :], (((1,), (0,)), ((), ())),
                         preferred_element_type=jnp.float32)
         + lax.dot_general(ftcat, w0[32:36, :], (((0,), (0,)), ((), ())),
                           preferred_element_type=jnp.float32)
         + b0_ref[...])
    h = jax.nn.relu(h)
    h = jax.nn.relu(
        lax.dot_general(h, w1_ref[...], (((1,), (0,)), ((), ())),
                        preferred_element_type=jnp.float32) + b1_ref[...])
    hm = jnp.maximum(h[0:_P, :], h[_P:2 * _P, :])
    prev = acc_ref[...]
    new = jnp.where(j == 0, hm, jnp.maximum(prev, hm))
    acc_ref[...] = new

    @pl.when(j == _K // 2 - 1)
    def _():
        x2 = jax.nn.relu(new)
        pooled = jnp.max(x2, axis=0, keepdims=True)
        out_ref[0] = (lax.dot_general(pooled, cw_ref[...],
                                      (((1,), (0,)), ((), ())),
                                      preferred_element_type=jnp.float32)
                      + cb_ref[...])


def _conv2_call(xg, ft, m2w0, m2b0, m2w1, m2b1, cw, cb, interpret=False):
    return pl.pallas_call(
        _conv2_body,
        grid=(_B, _K // 2),
        in_specs=[
            pl.BlockSpec((_P, 32), lambda b, j: (2 * j * _B + b, 0)),
            pl.BlockSpec((_P, 32), lambda b, j: ((2 * j + 1) * _B + b, 0)),
            pl.BlockSpec((1, 2, 4, _P), lambda b, j: (j, 0, 0, b)),
            pl.BlockSpec((36, 64), lambda b, j: (0, 0)),
            pl.BlockSpec((1, 64), lambda b, j: (0, 0)),
            pl.BlockSpec((64, 128), lambda b, j: (0, 0)),
            pl.BlockSpec((1, 128), lambda b, j: (0, 0)),
            pl.BlockSpec((128, _NUM_CLASSES), lambda b, j: (0, 0)),
            pl.BlockSpec((1, _NUM_CLASSES), lambda b, j: (0, 0)),
        ],
        out_specs=pl.BlockSpec((1, 1, _NUM_CLASSES), lambda b, j: (b, 0, 0)),
        out_shape=jax.ShapeDtypeStruct((_B, 1, _NUM_CLASSES), jnp.float32),
        scratch_shapes=[pltpu.VMEM((_P, 128), jnp.float32)],
        interpret=interpret,
    )(xg, xg, ft, m2w0, m2b0.reshape(1, 64), m2w1, m2b1.reshape(1, 128),
      cw, cb.reshape(1, _NUM_CLASSES)).reshape(_B, _NUM_CLASSES)


def kernel(pos, normal, batch, m1w0, m1b0, m1w1, m1b1, m2w0, m2b0, m2w1, m2b1,
           cw, cb):
    del batch
    idx = _knn_call(pos)
    src_km = idx.T.reshape(_E)
    table = jnp.concatenate(
        [pos, normal, jnp.zeros((_N, 10), jnp.float32)], axis=1)
    table_t = jnp.concatenate([pos.T, normal.T], axis=0)
    table_t = jnp.concatenate(
        [table_t, jnp.zeros((10, _N), jnp.float32)], axis=0)
    g = _sc_gather(table, src_km, chunk=4096)
    g_t = g.reshape(_K, _N, 16).transpose(0, 2, 1)
    x, ft = _conv1_call(g_t, table_t, m1w0, m1b0, m1w1, m1b1)
    xg = _sc_gather(x, src_km)
    return _conv2_call(xg, ft, m2w0, m2b0, m2w1, m2b1, cw, cb)


# speedup vs baseline: 1.2620x; 1.0614x over previous
import functools

import jax
import jax.numpy as jnp
from jax import lax
from jax.experimental import pallas as pl
from jax.experimental.pallas import tpu as pltpu
from jax.experimental.pallas import tpu_sc as plsc

_B, _P, _K = 32, 1024, 16
_N = _B * _P
_E = _N * _K
_NUM_CLASSES = 40
_RB = 512


def _knn_body(posr_ref, posc_ref, idx_ref, d2_ref):
    b = pl.program_id(0)
    pr = posr_ref[...]
    pc = posc_ref[...]
    sqr = jnp.sum(pr * pr, axis=1, keepdims=True)
    sqc = jnp.sum(pc * pc, axis=1)
    dots = lax.dot_general(pr, pc, (((1,), (1,)), ((), ())),
                           preferred_element_type=jnp.float32)
    d2_ref[...] = sqr + sqc[None, :] - 2.0 * dots
    colids = lax.broadcasted_iota(jnp.int32, (_RB, _P), 1)
    lanek = lax.broadcasted_iota(jnp.int32, (_RB, _K), 1)
    inf = jnp.float32(jnp.inf)

    def body(k, sels):
        d2 = d2_ref[...]
        m = jnp.min(d2, axis=1, keepdims=True)
        cand = jnp.where(d2 == m, colids, _P)
        sel = jnp.min(cand, axis=1, keepdims=True)
        d2_ref[...] = jnp.where(colids == sel, inf, d2)
        return jnp.where(lanek == k, sel, sels)

    sels = lax.fori_loop(0, _K, body, jnp.zeros((_RB, _K), jnp.int32))
    idx_ref[...] = sels + b * _P


def _knn_call(pos, interpret=False):
    nr = _P // _RB
    return pl.pallas_call(
        _knn_body,
        grid=(_B, nr),
        in_specs=[
            pl.BlockSpec((_RB, 3), lambda b, r: (b * nr + r, 0)),
            pl.BlockSpec((_P, 3), lambda b, r: (b, 0)),
        ],
        out_specs=pl.BlockSpec((_RB, _K), lambda b, r: (b * nr + r, 0)),
        out_shape=jax.ShapeDtypeStruct((_N, _K), jnp.int32),
        scratch_shapes=[pltpu.VMEM((_RB, _P), jnp.float32)],
        interpret=interpret,
    )(pos, pos)


def _sc_gather(table, idx, chunk=2048):
    _, d = table.shape
    m = idx.shape[0]
    nw = 32
    per_w = m // nw
    nch = per_w // chunk
    assert per_w % chunk == 0 and d % 16 == 0
    mesh = plsc.VectorSubcoreMesh(core_axis_name="c", subcore_axis_name="s")

    @functools.partial(
        pl.kernel, mesh=mesh,
        out_type=jax.ShapeDtypeStruct((m, d), jnp.float32),
        scratch_types=[
            pltpu.VMEM((chunk,), jnp.int32),
            pltpu.VMEM((chunk, d), jnp.float32),
            pltpu.SemaphoreType.DMA,
        ],
        compiler_params=pltpu.CompilerParams(use_tc_tiling_on_sc=False),
    )
    def k(table_hbm, idx_hbm, out_hbm, idx_v, rows_v, sem):
        wid = lax.axis_index("s") * 2 + lax.axis_index("c")

        def body(c, carry):
            base = wid * per_w + c * chunk
            pltpu.sync_copy(idx_hbm.at[pl.ds(base, chunk)], idx_v)
            pltpu.async_copy(table_hbm.at[idx_v], rows_v, sem).wait()
            pltpu.sync_copy(rows_v, out_hbm.at[pl.ds(base, chunk)])
            return carry

        lax.fori_loop(0, nch, body, 0)

    return k(table, idx)


def _psqrt(x):
    r = lax.rsqrt(x)
    r = r * (1.5 - 0.5 * x * r * r)
    r = r * (1.5 - 0.5 * x * r * r)
    return x * r

_ATAN_C = (0.9999999880821423, -0.333331207776302, 0.1999371609301849,
           -0.14213195870325876, 0.10681419898649704, -0.07596807640005956,
           0.04385557421611047, -0.016827433863723106, 0.003049964511658644)


def _atan2_nn(y, x):
    ax = jnp.abs(x)
    num = jnp.minimum(y, ax)
    den = jnp.maximum(y, ax)
    r = lax.reciprocal(den)
    r = r * (2.0 - den * r)
    r = r * (2.0 - den * r)
    q = num * r
    s = q * q
    acc = jnp.float32(_ATAN_C[-1])
    for c in _ATAN_C[-2::-1]:
        acc = acc * s + jnp.float32(c)
    base = acc * q
    res = jnp.where(y > ax, jnp.float32(jnp.pi / 2) - base, base)
    return jnp.where(x < 0, jnp.float32(jnp.pi) - res, res)


def _angle(v1x, v1y, v1z, v2x, v2y, v2z):
    cx = v1y * v2z - v1z * v2y
    cy = v1z * v2x - v1x * v2z
    cz = v1x * v2y - v1y * v2x
    cn2 = cx * cx + cy * cy + cz * cz
    cn = jnp.where(cn2 == 0, 0.0, _psqrt(jnp.where(cn2 == 0, 1.0, cn2)))
    dot = v1x * v2x + v1y * v2y + v1z * v2z
    both = (cn == 0) & (dot == 0)
    return _atan2_nn(jnp.where(both, 0.0, cn), jnp.where(both, 1.0, dot))


def _ppf_rows(git, gjt):
    dx = gjt[0:1, :] - git[0:1, :]
    dy = gjt[1:2, :] - git[1:2, :]
    dz = gjt[2:3, :] - git[2:3, :]
    nix, niy, niz = git[3:4, :], git[4:5, :], git[5:6, :]
    njx, njy, njz = gjt[3:4, :], gjt[4:5, :], gjt[5:6, :]
    d2s = dx * dx + dy * dy + dz * dz
    dn = jnp.where(d2s == 0, 0.0, _psqrt(jnp.where(d2s == 0, 1.0, d2s)))
    a1 = _angle(nix, niy, niz, dx, dy, dz)
    a2 = _angle(njx, njy, njz, dx, dy, dz)
    a3 = _angle(nix, niy, niz, njx, njy, njz)
    return dn, a1, a2, a3


def _conv1_body(gj0_ref, gj1_ref, gj2_ref, gj3_ref, git_ref,
                w0_ref, b0_ref, w1_ref, b1_ref, x_ref, ft_ref, acc_ref):
    j = pl.program_id(1)
    git = git_ref[...]
    fts = []
    for i, gj_ref in enumerate((gj0_ref, gj1_ref, gj2_ref, gj3_ref)):
        dn, a1, a2, a3 = _ppf_rows(git, gj_ref[0])
        fts.append(jnp.concatenate([dn, a1, a2, a3], axis=0))
        ft_ref[0, i] = fts[i]
    ftcat = jnp.concatenate(fts, axis=1)
    h = jax.nn.relu(
        lax.dot_general(ftcat, w0_ref[...], (((0,), (0,)), ((), ())),
                        preferred_element_type=jnp.float32) + b0_ref[...])
    h = jax.nn.relu(
        lax.dot_general(h, w1_ref[...], (((1,), (0,)), ((), ())),
                        preferred_element_type=jnp.float32) + b1_ref[...])
    hm = jnp.maximum(jnp.maximum(h[0:_P, :], h[_P:2 * _P, :]),
                     jnp.maximum(h[2 * _P:3 * _P, :], h[3 * _P:4 * _P, :]))
    prev = acc_ref[...]
    new = jnp.where(j == 0, hm, jnp.maximum(prev, hm))
    acc_ref[...] = new

    @pl.when(j == _K // 4 - 1)
    def _():
        x_ref[...] = jax.nn.relu(new)


def _conv1_call(g, table_t, m1w0, m1b0, m1w1, m1b1, interpret=False):
    return pl.pallas_call(
        _conv1_body,
        grid=(_B, _K // 4),
        in_specs=[
            pl.BlockSpec((1, 16, _P), lambda b, j: (4 * j, 0, b)),
            pl.BlockSpec((1, 16, _P), lambda b, j: (4 * j + 1, 0, b)),
            pl.BlockSpec((1, 16, _P), lambda b, j: (4 * j + 2, 0, b)),
            pl.BlockSpec((1, 16, _P), lambda b, j: (4 * j + 3, 0, b)),
            pl.BlockSpec((16, _P), lambda b, j: (0, b)),
            pl.BlockSpec((4, 16), lambda b, j: (0, 0)),
            pl.BlockSpec((1, 16), lambda b, j: (0, 0)),
            pl.BlockSpec((16, 32), lambda b, j: (0, 0)),
            pl.BlockSpec((1, 32), lambda b, j: (0, 0)),
        ],
        out_specs=[
            pl.BlockSpec((_P, 32), lambda b, j: (b, 0)),
            pl.BlockSpec((1, 4, 4, _P), lambda b, j: (j, 0, 0, b)),
        ],
        out_shape=[
            jax.ShapeDtypeStruct((_N, 32), jnp.float32),
            jax.ShapeDtypeStruct((_K // 4, 4, 4, _N), jnp.float32),
        ],
        scratch_shapes=[pltpu.VMEM((_P, 32), jnp.float32)],
        interpret=interpret,
    )(g, g, g, g, table_t, m1w0, m1b0.reshape(1, 16), m1w1,
      m1b1.reshape(1, 32))


def _conv2_body(xg0_ref, xg1_ref, xg2_ref, xg3_ref, ft_ref, w0_ref, b0_ref,
                w1_ref, b1_ref, cw_ref, cb_ref, out_ref, acc_ref):
    j = pl.program_id(1)
    w0 = w0_ref[...]
    xgcat = jnp.concatenate(
        [xg0_ref[...], xg1_ref[...], xg2_ref[...], xg3_ref[...]], axis=0)
    ftcat = jnp.concatenate(
        [ft_ref[0, 0], ft_ref[0, 1], ft_ref[0, 2], ft_ref[0, 3]], axis=1)
    h = (lax.dot_general(xgcat, w0[0:32, :], (((1,), (0,)), ((), ())),
                         preferred_element_type=jnp.float32)
         + lax.dot_general(ftcat, w0[32:36, :], (((0,), (0,)), ((), ())),
                           preferred_element_type=jnp.float32)
         + b0_ref[...])
    h = jax.nn.relu(h)
    h = jax.nn.relu(
        lax.dot_general(h, w1_ref[...], (((1,), (0,)), ((), ())),
                        preferred_element_type=jnp.float32) + b1_ref[...])
    hm = jnp.maximum(jnp.maximum(h[0:_P, :], h[_P:2 * _P, :]),
                     jnp.maximum(h[2 * _P:3 * _P, :], h[3 * _P:4 * _P, :]))
    prev = acc_ref[...]
    new = jnp.where(j == 0, hm, jnp.maximum(prev, hm))
    acc_ref[...] = new

    @pl.when(j == _K // 4 - 1)
    def _():
        x2 = jax.nn.relu(new)
        pooled = jnp.max(x2, axis=0, keepdims=True)
        out_ref[0] = (lax.dot_general(pooled, cw_ref[...],
                                      (((1,), (0,)), ((), ())),
                                      preferred_element_type=jnp.float32)
                      + cb_ref[...])


def _conv2_call(xg, ft, m2w0, m2b0, m2w1, m2b1, cw, cb, interpret=False):
    return pl.pallas_call(
        _conv2_body,
        grid=(_B, _K // 4),
        in_specs=[
            pl.BlockSpec((_P, 32), lambda b, j: (4 * j * _B + b, 0)),
            pl.BlockSpec((_P, 32), lambda b, j: ((4 * j + 1) * _B + b, 0)),
            pl.BlockSpec((_P, 32), lambda b, j: ((4 * j + 2) * _B + b, 0)),
            pl.BlockSpec((_P, 32), lambda b, j: ((4 * j + 3) * _B + b, 0)),
            pl.BlockSpec((1, 4, 4, _P), lambda b, j: (j, 0, 0, b)),
            pl.BlockSpec((36, 64), lambda b, j: (0, 0)),
            pl.BlockSpec((1, 64), lambda b, j: (0, 0)),
            pl.BlockSpec((64, 128), lambda b, j: (0, 0)),
            pl.BlockSpec((1, 128), lambda b, j: (0, 0)),
            pl.BlockSpec((128, _NUM_CLASSES), lambda b, j: (0, 0)),
            pl.BlockSpec((1, _NUM_CLASSES), lambda b, j: (0, 0)),
        ],
        out_specs=pl.BlockSpec((1, 1, _NUM_CLASSES), lambda b, j: (b, 0, 0)),
        out_shape=jax.ShapeDtypeStruct((_B, 1, _NUM_CLASSES), jnp.float32),
        scratch_shapes=[pltpu.VMEM((_P, 128), jnp.float32)],
        interpret=interpret,
    )(xg, xg, xg, xg, ft, m2w0, m2b0.reshape(1, 64), m2w1, m2b1.reshape(1, 128),
      cw, cb.reshape(1, _NUM_CLASSES)).reshape(_B, _NUM_CLASSES)


def kernel(pos, normal, batch, m1w0, m1b0, m1w1, m1b1, m2w0, m2b0, m2w1, m2b1,
           cw, cb):
    del batch
    idx = _knn_call(pos)
    src_km = idx.T.reshape(_E)
    table = jnp.concatenate(
        [pos, normal, jnp.zeros((_N, 10), jnp.float32)], axis=1)
    table_t = jnp.concatenate([pos.T, normal.T], axis=0)
    table_t = jnp.concatenate(
        [table_t, jnp.zeros((10, _N), jnp.float32)], axis=0)
    g = _sc_gather(table, src_km, chunk=4096)
    g_t = g.reshape(_K, _N, 16).transpose(0, 2, 1)
    x, ft = _conv1_call(g_t, table_t, m1w0, m1b0, m1w1, m1b1)
    xg = _sc_gather(x, src_km)
    return _conv2_call(xg, ft, m2w0, m2b0, m2w1, m2b1, cw, cb)
